# Initial kernel scaffold; baseline (speedup 1.0000x reference)
import jax
import jax.numpy as jnp
from jax.experimental import pallas as pl


def kernel(x, edge_index, W1, b1, W2, b2):
    raise NotImplementedError("write your pallas kernel here")


# speedup vs baseline: 41.5219x; 41.5219x over previous
import functools

import jax
import jax.numpy as jnp
from jax import lax
from jax.experimental import pallas as pl
from jax.experimental.pallas import tpu as pltpu
from jax.experimental.pallas import tpu_sc as plsc

_NC = 2
_NS = 16
_NW = _NC * _NS
_IDX = 128
_D = 16


def _mesh():
    return plsc.VectorSubcoreMesh(core_axis_name="c", subcore_axis_name="s")


_SC_PARAMS = pltpu.CompilerParams(use_tc_tiling_on_sc=False)


def _deg_body(n_pad, ch, dst_hbm, deg_out, idx_v, ones_v, zero_v, deg_sh, sem):
    c = lax.axis_index("c")
    s = lax.axis_index("s")
    w = c * _NS + s
    zc = n_pad // _NS
    for j in range(8):
        ones_v[pl.ds(16 * j, 16)] = jnp.ones((16,), jnp.float32)
    for j in range(zc // 16):
        zero_v[pl.ds(16 * j, 16)] = jnp.zeros((16,), jnp.float32)
    pltpu.sync_copy(zero_v, deg_sh.at[pl.ds(s * zc, zc)])
    pltpu.sync_copy(dst_hbm.at[w], idx_v)
    plsc.subcore_barrier()

    def step(j, carry):
        pltpu.sync_copy(ones_v, deg_sh.at[idx_v.at[j]], add=True)
        return carry

    lax.fori_loop(0, ch, step, 0)
    plsc.subcore_barrier()
    pltpu.sync_copy(deg_sh.at[pl.ds(s * zc, zc)],
                    deg_out.at[c, pl.ds(s * zc, zc)])


def _msg_body(n_pad, ch, src_hbm, dst_hbm, g_hbm, out_hbm,
              isrc_v, idst_v, vals_v, s_sh, sem):
    c = lax.axis_index("c")
    s = lax.axis_index("s")
    w = c * _NS + s
    zc = n_pad // _NS
    pltpu.sync_copy(g_hbm.at[pl.ds(s * zc, zc)], s_sh.at[pl.ds(s * zc, zc)])
    pltpu.sync_copy(src_hbm.at[w], isrc_v)
    pltpu.sync_copy(dst_hbm.at[w], idst_v)
    plsc.subcore_barrier()

    def step(j, carry):
        pltpu.async_copy(g_hbm.at[isrc_v.at[j]], vals_v, sem).wait()
        pltpu.sync_copy(vals_v, s_sh.at[idst_v.at[j]], add=True)
        return carry

    lax.fori_loop(0, ch, step, 0)
    plsc.subcore_barrier()
    pltpu.sync_copy(s_sh.at[pl.ds(s * zc, zc)],
                    out_hbm.at[c, pl.ds(s * zc, zc)])


def _g_body(x_ref, w1_ref, d0_ref, d1_ref, g_ref):
    h = jnp.dot(x_ref[...], w1_ref[...], preferred_element_type=jnp.float32)
    deg = d0_ref[...] + d1_ref[...] + 1.0
    dinv = lax.rsqrt(deg)[:, None]
    colmask = (lax.broadcasted_iota(jnp.int32, h.shape, 1) == 3)
    g_ref[...] = dinv * (h + colmask.astype(jnp.float32))


def _out_body(s0_ref, s1_ref, g_ref, w2_ref, b1_ref, b2_ref, o_ref):
    g = g_ref[...]
    st = s0_ref[...] + s1_ref[...] - g
    acc = g[:, 3:4] * st + b1_ref[...]
    o_ref[...] = jnp.dot(acc, w2_ref[...],
                         preferred_element_type=jnp.float32) + b2_ref[...]


def kernel(x, edge_index, W1, b1, W2, b2):
    n, f = x.shape
    h = W1.shape[1]
    cdim = W2.shape[1]
    e = edge_index.shape[1]

    blk = 1024
    n_pad = -(-n // blk) * blk
    grid = n_pad // blk

    per_tile = -(-e // _NW)
    pt = -(-per_tile // _IDX) * _IDX
    ch = pt // _IDX
    e_pad = _NW * pt

    src = edge_index[0].astype(jnp.int32)
    dst = edge_index[1].astype(jnp.int32)
    src_p = jnp.concatenate(
        [src, jnp.zeros((e_pad - e,), jnp.int32)]).reshape(_NW, ch, _IDX)
    dst_p = jnp.concatenate(
        [dst, jnp.full((e_pad - e,), n, jnp.int32)]).reshape(_NW, ch, _IDX)

    x_p = jnp.pad(x, ((0, n_pad - n), (0, 0)))
    w1_p = jnp.pad(W1, ((0, 0), (0, _D - h)))
    w2_p = jnp.pad(W2, ((0, _D - h), (0, 0)))
    b1_p = jnp.pad(b1, (0, _D - h)).reshape(1, _D)
    b2_p = b2.reshape(1, cdim)

    deg_call = functools.partial(
        pl.kernel,
        out_type=jax.ShapeDtypeStruct((_NC, n_pad), jnp.float32),
        mesh=_mesh(),
        scratch_types=[
            pltpu.VMEM((ch, _IDX), jnp.int32),
            pltpu.VMEM((_IDX,), jnp.float32),
            pltpu.VMEM((n_pad // _NS,), jnp.float32),
            pltpu.VMEM_SHARED((n_pad,), jnp.float32),
            pltpu.SemaphoreType.DMA,
        ],
        compiler_params=_SC_PARAMS,
    )(functools.partial(_deg_body, n_pad, ch))
    deg = deg_call(dst_p)

    g_arr = pl.pallas_call(
        _g_body,
        grid=(grid,),
        in_specs=[
            pl.BlockSpec((blk, f), lambda i: (i, 0)),
            pl.BlockSpec((f, _D), lambda i: (0, 0)),
            pl.BlockSpec((blk,), lambda i: (i,)),
            pl.BlockSpec((blk,), lambda i: (i,)),
        ],
        out_specs=pl.BlockSpec((blk, _D), lambda i: (i, 0)),
        out_shape=jax.ShapeDtypeStruct((n_pad, _D), jnp.float32),
    )(x_p, w1_p, deg[0], deg[1])

    msg_call = functools.partial(
        pl.kernel,
        out_type=jax.ShapeDtypeStruct((_NC, n_pad, _D), jnp.float32),
        mesh=_mesh(),
        scratch_types=[
            pltpu.VMEM((ch, _IDX), jnp.int32),
            pltpu.VMEM((ch, _IDX), jnp.int32),
            pltpu.VMEM((_IDX, _D), jnp.float32),
            pltpu.VMEM_SHARED((n_pad, _D), jnp.float32),
            pltpu.SemaphoreType.DMA,
        ],
        compiler_params=_SC_PARAMS,
    )(functools.partial(_msg_body, n_pad, ch))
    s_part = msg_call(src_p, dst_p, g_arr)

    out = pl.pallas_call(
        _out_body,
        grid=(grid,),
        in_specs=[
            pl.BlockSpec((blk, _D), lambda i: (i, 0)),
            pl.BlockSpec((blk, _D), lambda i: (i, 0)),
            pl.BlockSpec((blk, _D), lambda i: (i, 0)),
            pl.BlockSpec((_D, cdim), lambda i: (0, 0)),
            pl.BlockSpec((1, _D), lambda i: (0, 0)),
            pl.BlockSpec((1, cdim), lambda i: (0, 0)),
        ],
        out_specs=pl.BlockSpec((blk, cdim), lambda i: (i, 0)),
        out_shape=jax.ShapeDtypeStruct((n_pad, cdim), jnp.float32),
    )(s_part[0], s_part[1], g_arr, w2_p, b1_p, b2_p)

    return out[:n]


# speedup vs baseline: 48.0850x; 1.1581x over previous
import functools

import jax
import jax.numpy as jnp
from jax import lax
from jax.experimental import pallas as pl
from jax.experimental.pallas import tpu as pltpu
from jax.experimental.pallas import tpu_sc as plsc

_NC = 2
_NS = 16
_NW = _NC * _NS
_IDX = 128
_D = 16


def _mesh():
    return plsc.VectorSubcoreMesh(core_axis_name="c", subcore_axis_name="s")


_SC_PARAMS = pltpu.CompilerParams(use_tc_tiling_on_sc=False)


def _deg_body(n_pad, ch, dst_hbm, deg_out, idx_v, ones_v, zero_v, deg_sh, sem):
    c = lax.axis_index("c")
    s = lax.axis_index("s")
    w = c * _NS + s
    zc = n_pad // _NS
    for j in range(8):
        ones_v[pl.ds(16 * j, 16)] = jnp.ones((16,), jnp.float32)
    for j in range(zc // 16):
        zero_v[pl.ds(16 * j, 16)] = jnp.zeros((16,), jnp.float32)
    pltpu.sync_copy(zero_v, deg_sh.at[pl.ds(s * zc, zc)])
    pltpu.sync_copy(dst_hbm.at[w], idx_v)
    plsc.subcore_barrier()

    def step(j, carry):
        pltpu.sync_copy(ones_v, deg_sh.at[idx_v.at[j]], add=True)
        return carry

    lax.fori_loop(0, ch, step, 0)
    plsc.subcore_barrier()
    pltpu.sync_copy(deg_sh.at[pl.ds(s * zc, zc)],
                    deg_out.at[c, pl.ds(s * zc, zc)])


def _msg_body(n_pad, ch, src_hbm, dst_hbm, g_hbm, out_hbm,
              isrc_v, idst_v, vals0_v, vals1_v, s_sh, sem0, sem1):
    c = lax.axis_index("c")
    s = lax.axis_index("s")
    w = c * _NS + s
    zc = n_pad // _NS
    pltpu.sync_copy(g_hbm.at[pl.ds(s * zc, zc)], s_sh.at[pl.ds(s * zc, zc)])
    pltpu.sync_copy(src_hbm.at[w], isrc_v)
    pltpu.sync_copy(dst_hbm.at[w], idst_v)
    plsc.subcore_barrier()

    pltpu.async_copy(g_hbm.at[isrc_v.at[0]], vals0_v, sem0)

    def step(jj, carry):
        j0 = 2 * jj
        pltpu.async_copy(g_hbm.at[isrc_v.at[j0 + 1]], vals1_v, sem1)
        pltpu.make_async_copy(g_hbm.at[isrc_v.at[j0]], vals0_v, sem0).wait()
        pltpu.sync_copy(vals0_v, s_sh.at[idst_v.at[j0]], add=True)

        @pl.when(j0 + 2 < ch)
        def _():
            pltpu.async_copy(g_hbm.at[isrc_v.at[j0 + 2]], vals0_v, sem0)

        pltpu.make_async_copy(g_hbm.at[isrc_v.at[j0 + 1]], vals1_v, sem1).wait()
        pltpu.sync_copy(vals1_v, s_sh.at[idst_v.at[j0 + 1]], add=True)
        return carry

    lax.fori_loop(0, ch // 2, step, 0)
    plsc.subcore_barrier()
    pltpu.sync_copy(s_sh.at[pl.ds(s * zc, zc)],
                    out_hbm.at[c, pl.ds(s * zc, zc)])


def _g_body(x_ref, w1_ref, d0_ref, d1_ref, g_ref):
    h = jnp.dot(x_ref[...], w1_ref[...], preferred_element_type=jnp.float32)
    deg = d0_ref[...] + d1_ref[...] + 1.0
    dinv = lax.rsqrt(deg)[:, None]
    colmask = (lax.broadcasted_iota(jnp.int32, h.shape, 1) == 3)
    g_ref[...] = dinv * (h + colmask.astype(jnp.float32))


def _out_body(s0_ref, s1_ref, g_ref, w2_ref, b1_ref, b2_ref, o_ref):
    g = g_ref[...]
    st = s0_ref[...] + s1_ref[...] - g
    acc = g[:, 3:4] * st + b1_ref[...]
    o_ref[...] = jnp.dot(acc, w2_ref[...],
                         preferred_element_type=jnp.float32) + b2_ref[...]


def kernel(x, edge_index, W1, b1, W2, b2):
    n, f = x.shape
    h = W1.shape[1]
    cdim = W2.shape[1]
    e = edge_index.shape[1]

    blk = 1024
    n_pad = -(-n // blk) * blk
    grid = n_pad // blk

    per_tile = -(-e // _NW)
    pt = -(-per_tile // (2 * _IDX)) * (2 * _IDX)
    ch = pt // _IDX
    e_pad = _NW * pt

    src = edge_index[0].astype(jnp.int32)
    dst = edge_index[1].astype(jnp.int32)
    src_p = jnp.concatenate(
        [src, jnp.zeros((e_pad - e,), jnp.int32)]).reshape(_NW, ch, _IDX)
    dst_p = jnp.concatenate(
        [dst, jnp.full((e_pad - e,), n, jnp.int32)]).reshape(_NW, ch, _IDX)

    x_p = jnp.pad(x, ((0, n_pad - n), (0, 0)))
    w1_p = jnp.pad(W1, ((0, 0), (0, _D - h)))
    w2_p = jnp.pad(W2, ((0, _D - h), (0, 0)))
    b1_p = jnp.pad(b1, (0, _D - h)).reshape(1, _D)
    b2_p = b2.reshape(1, cdim)

    deg_call = functools.partial(
        pl.kernel,
        out_type=jax.ShapeDtypeStruct((_NC, n_pad), jnp.float32),
        mesh=_mesh(),
        scratch_types=[
            pltpu.VMEM((ch, _IDX), jnp.int32),
            pltpu.VMEM((_IDX,), jnp.float32),
            pltpu.VMEM((n_pad // _NS,), jnp.float32),
            pltpu.VMEM_SHARED((n_pad,), jnp.float32),
            pltpu.SemaphoreType.DMA,
        ],
        compiler_params=_SC_PARAMS,
    )(functools.partial(_deg_body, n_pad, ch))
    deg = deg_call(dst_p)

    g_arr = pl.pallas_call(
        _g_body,
        grid=(grid,),
        in_specs=[
            pl.BlockSpec((blk, f), lambda i: (i, 0)),
            pl.BlockSpec((f, _D), lambda i: (0, 0)),
            pl.BlockSpec((blk,), lambda i: (i,)),
            pl.BlockSpec((blk,), lambda i: (i,)),
        ],
        out_specs=pl.BlockSpec((blk, _D), lambda i: (i, 0)),
        out_shape=jax.ShapeDtypeStruct((n_pad, _D), jnp.float32),
    )(x_p, w1_p, deg[0], deg[1])

    msg_call = functools.partial(
        pl.kernel,
        out_type=jax.ShapeDtypeStruct((_NC, n_pad, _D), jnp.float32),
        mesh=_mesh(),
        scratch_types=[
            pltpu.VMEM((ch, _IDX), jnp.int32),
            pltpu.VMEM((ch, _IDX), jnp.int32),
            pltpu.VMEM((_IDX, _D), jnp.float32),
            pltpu.VMEM((_IDX, _D), jnp.float32),
            pltpu.VMEM_SHARED((n_pad, _D), jnp.float32),
            pltpu.SemaphoreType.DMA,
            pltpu.SemaphoreType.DMA,
        ],
        compiler_params=_SC_PARAMS,
    )(functools.partial(_msg_body, n_pad, ch))
    s_part = msg_call(src_p, dst_p, g_arr)

    out = pl.pallas_call(
        _out_body,
        grid=(grid,),
        in_specs=[
            pl.BlockSpec((blk, _D), lambda i: (i, 0)),
            pl.BlockSpec((blk, _D), lambda i: (i, 0)),
            pl.BlockSpec((blk, _D), lambda i: (i, 0)),
            pl.BlockSpec((_D, cdim), lambda i: (0, 0)),
            pl.BlockSpec((1, _D), lambda i: (0, 0)),
            pl.BlockSpec((1, cdim), lambda i: (0, 0)),
        ],
        out_specs=pl.BlockSpec((blk, cdim), lambda i: (i, 0)),
        out_shape=jax.ShapeDtypeStruct((n_pad, cdim), jnp.float32),
    )(s_part[0], s_part[1], g_arr, w2_p, b1_p, b2_p)

    return out[:n]


# speedup vs baseline: 49.7712x; 1.0351x over previous
import functools

import jax
import jax.numpy as jnp
from jax import lax
from jax.experimental import pallas as pl
from jax.experimental.pallas import tpu as pltpu
from jax.experimental.pallas import tpu_sc as plsc

_NC = 2
_NS = 16
_NW = _NC * _NS
_IDX = 128
_D = 8


def _mesh():
    return plsc.VectorSubcoreMesh(core_axis_name="c", subcore_axis_name="s")


_SC_PARAMS = pltpu.CompilerParams(use_tc_tiling_on_sc=False)


def _deg_body(n_pad, ch, dst_hbm, deg_out, idx_v, ones_v, zero_v, deg_sh, sem):
    c = lax.axis_index("c")
    s = lax.axis_index("s")
    w = c * _NS + s
    zc = n_pad // _NS
    for j in range(8):
        ones_v[pl.ds(16 * j, 16)] = jnp.ones((16,), jnp.float32)
    for j in range(zc // 16):
        zero_v[pl.ds(16 * j, 16)] = jnp.zeros((16,), jnp.float32)
    pltpu.sync_copy(zero_v, deg_sh.at[pl.ds(s * zc, zc)])
    pltpu.sync_copy(dst_hbm.at[w], idx_v)
    plsc.subcore_barrier()

    def step(j, carry):
        pltpu.sync_copy(ones_v, deg_sh.at[idx_v.at[j]], add=True)
        return carry

    lax.fori_loop(0, ch, step, 0)
    plsc.subcore_barrier()
    pltpu.sync_copy(deg_sh.at[pl.ds(s * zc, zc)],
                    deg_out.at[c, pl.ds(s * zc, zc)])


def _msg_body(n_pad, ch, src_hbm, dst_hbm, g_hbm, out_hbm,
              isrc_v, idst_v, vals0_v, vals1_v, s_sh, sem0, sem1):
    c = lax.axis_index("c")
    s = lax.axis_index("s")
    w = c * _NS + s
    zc = n_pad // _NS
    pltpu.sync_copy(g_hbm.at[pl.ds(s * zc, zc)], s_sh.at[pl.ds(s * zc, zc)])
    pltpu.sync_copy(src_hbm.at[w], isrc_v)
    pltpu.sync_copy(dst_hbm.at[w], idst_v)
    plsc.subcore_barrier()

    pltpu.async_copy(g_hbm.at[isrc_v.at[0]], vals0_v, sem0)

    def step(jj, carry):
        j0 = 2 * jj
        pltpu.async_copy(g_hbm.at[isrc_v.at[j0 + 1]], vals1_v, sem1)
        pltpu.make_async_copy(g_hbm.at[isrc_v.at[j0]], vals0_v, sem0).wait()
        pltpu.sync_copy(vals0_v, s_sh.at[idst_v.at[j0]], add=True)

        @pl.when(j0 + 2 < ch)
        def _():
            pltpu.async_copy(g_hbm.at[isrc_v.at[j0 + 2]], vals0_v, sem0)

        pltpu.make_async_copy(g_hbm.at[isrc_v.at[j0 + 1]], vals1_v, sem1).wait()
        pltpu.sync_copy(vals1_v, s_sh.at[idst_v.at[j0 + 1]], add=True)
        return carry

    lax.fori_loop(0, ch // 2, step, 0)
    plsc.subcore_barrier()
    pltpu.sync_copy(s_sh.at[pl.ds(s * zc, zc)],
                    out_hbm.at[c, pl.ds(s * zc, zc)])


def _g_body(x_ref, w1_ref, d0_ref, d1_ref, g_ref):
    h = jnp.dot(x_ref[...], w1_ref[...], preferred_element_type=jnp.float32)
    deg = d0_ref[...] + d1_ref[...] + 1.0
    dinv = lax.rsqrt(deg)[:, None]
    colmask = (lax.broadcasted_iota(jnp.int32, h.shape, 1) == 3)
    g_ref[...] = dinv * (h + colmask.astype(jnp.float32))


def _out_body(s0_ref, s1_ref, g_ref, w2_ref, b1_ref, b2_ref, o_ref):
    g = g_ref[...]
    st = s0_ref[...] + s1_ref[...] - g
    acc = g[:, 3:4] * st + b1_ref[...]
    o_ref[...] = jnp.dot(acc, w2_ref[...],
                         preferred_element_type=jnp.float32) + b2_ref[...]


def kernel(x, edge_index, W1, b1, W2, b2):
    n, f = x.shape
    h = W1.shape[1]
    cdim = W2.shape[1]
    e = edge_index.shape[1]

    blk = 1024
    n_pad = -(-n // blk) * blk
    grid = n_pad // blk

    per_tile = -(-e // _NW)
    pt = -(-per_tile // (2 * _IDX)) * (2 * _IDX)
    ch = pt // _IDX
    e_pad = _NW * pt

    src = edge_index[0].astype(jnp.int32)
    dst = edge_index[1].astype(jnp.int32)
    src_p = jnp.concatenate(
        [src, jnp.zeros((e_pad - e,), jnp.int32)]).reshape(_NW, ch, _IDX)
    dst_p = jnp.concatenate(
        [dst, jnp.full((e_pad - e,), n, jnp.int32)]).reshape(_NW, ch, _IDX)

    x_p = jnp.pad(x, ((0, n_pad - n), (0, 0)))
    w1_p = jnp.pad(W1, ((0, 0), (0, _D - h)))
    w2_p = jnp.pad(W2, ((0, _D - h), (0, 0)))
    b1_p = jnp.pad(b1, (0, _D - h)).reshape(1, _D)
    b2_p = b2.reshape(1, cdim)

    deg_call = functools.partial(
        pl.kernel,
        out_type=jax.ShapeDtypeStruct((_NC, n_pad), jnp.float32),
        mesh=_mesh(),
        scratch_types=[
            pltpu.VMEM((ch, _IDX), jnp.int32),
            pltpu.VMEM((_IDX,), jnp.float32),
            pltpu.VMEM((n_pad // _NS,), jnp.float32),
            pltpu.VMEM_SHARED((n_pad,), jnp.float32),
            pltpu.SemaphoreType.DMA,
        ],
        compiler_params=_SC_PARAMS,
    )(functools.partial(_deg_body, n_pad, ch))
    deg = deg_call(dst_p)

    g_arr = pl.pallas_call(
        _g_body,
        grid=(grid,),
        in_specs=[
            pl.BlockSpec((blk, f), lambda i: (i, 0)),
            pl.BlockSpec((f, _D), lambda i: (0, 0)),
            pl.BlockSpec((blk,), lambda i: (i,)),
            pl.BlockSpec((blk,), lambda i: (i,)),
        ],
        out_specs=pl.BlockSpec((blk, _D), lambda i: (i, 0)),
        out_shape=jax.ShapeDtypeStruct((n_pad, _D), jnp.float32),
    )(x_p, w1_p, deg[0], deg[1])

    msg_call = functools.partial(
        pl.kernel,
        out_type=jax.ShapeDtypeStruct((_NC, n_pad, _D), jnp.float32),
        mesh=_mesh(),
        scratch_types=[
            pltpu.VMEM((ch, _IDX), jnp.int32),
            pltpu.VMEM((ch, _IDX), jnp.int32),
            pltpu.VMEM((_IDX, _D), jnp.float32),
            pltpu.VMEM((_IDX, _D), jnp.float32),
            pltpu.VMEM_SHARED((n_pad, _D), jnp.float32),
            pltpu.SemaphoreType.DMA,
            pltpu.SemaphoreType.DMA,
        ],
        compiler_params=_SC_PARAMS,
    )(functools.partial(_msg_body, n_pad, ch))
    s_part = msg_call(src_p, dst_p, g_arr)

    out = pl.pallas_call(
        _out_body,
        grid=(grid,),
        in_specs=[
            pl.BlockSpec((blk, _D), lambda i: (i, 0)),
            pl.BlockSpec((blk, _D), lambda i: (i, 0)),
            pl.BlockSpec((blk, _D), lambda i: (i, 0)),
            pl.BlockSpec((_D, cdim), lambda i: (0, 0)),
            pl.BlockSpec((1, _D), lambda i: (0, 0)),
            pl.BlockSpec((1, cdim), lambda i: (0, 0)),
        ],
        out_specs=pl.BlockSpec((blk, cdim), lambda i: (i, 0)),
        out_shape=jax.ShapeDtypeStruct((n_pad, cdim), jnp.float32),
    )(s_part[0], s_part[1], g_arr, w2_p, b1_p, b2_p)

    return out[:n]


# speedup vs baseline: 58.5472x; 1.1763x over previous
import functools

import jax
import jax.numpy as jnp
from jax import lax
from jax.experimental import pallas as pl
from jax.experimental.pallas import tpu as pltpu
from jax.experimental.pallas import tpu_sc as plsc

_NC = 2
_NS = 16
_NW = _NC * _NS
_IDX = 128
_D = 16


def _mesh():
    return plsc.VectorSubcoreMesh(core_axis_name="c", subcore_axis_name="s")


_SC_PARAMS = pltpu.CompilerParams(use_tc_tiling_on_sc=False,
                                  needs_layout_passes=False)


def _newton_rsqrt(v):
    i = plsc.bitcast(v, jnp.int32)
    i = 0x5F3759DF - lax.shift_right_logical(i, 1)
    y = plsc.bitcast(i, jnp.float32)
    half = 0.5 * v
    for _ in range(3):
        y = y * (1.5 - half * y * y)
    return y


def _fused_body(n_pad, chd, ch, dstd_hbm, src_hbm, dst_hbm, hp_hbm,
                s_out, dinv_out,
                idxd_v, isrc_v, idst_v, ones_v, zero_v, hp_v, g_v,
                deg_v, dinv_v, vals0_v, vals1_v,
                deg_sh, g_sh, s_sh, sem0, sem1):
    c = lax.axis_index("c")
    s = lax.axis_index("s")
    w = c * _NS + s
    zc = n_pad // _NS

    for j in range(8):
        ones_v[pl.ds(16 * j, 16)] = jnp.ones((16,), jnp.float32)
    for j in range(zc // 16):
        zero_v[pl.ds(16 * j, 16)] = jnp.zeros((16,), jnp.float32)
    pltpu.sync_copy(zero_v, deg_sh.at[pl.ds(s * zc, zc)])
    pltpu.sync_copy(dstd_hbm.at[s], idxd_v)
    pltpu.sync_copy(src_hbm.at[w], isrc_v)
    pltpu.sync_copy(dst_hbm.at[w], idst_v)
    pltpu.sync_copy(hp_hbm.at[pl.ds(s * zc, zc)], hp_v)
    plsc.subcore_barrier()

    def dstep(j, carry):
        pltpu.sync_copy(ones_v, deg_sh.at[idxd_v.at[j]], add=True)
        return carry

    lax.fori_loop(0, chd, dstep, 0)
    plsc.subcore_barrier()

    pltpu.sync_copy(deg_sh.at[pl.ds(s * zc, zc)], deg_v)

    def nstep(k, carry):
        d16 = deg_v[pl.ds(16 * k, 16)] + 1.0
        dinv_v[pl.ds(16 * k, 16)] = _newton_rsqrt(d16)
        return carry

    lax.fori_loop(0, zc // 16, nstep, 0)

    def gstep(k, carry):
        d16 = dinv_v[pl.ds(16 * k, 16)]
        for j in range(16):
            r = 16 * k + j
            g_v[r, :] = hp_v[r, :] * d16[j]
        return carry

    lax.fori_loop(0, zc // 16, gstep, 0)
    pltpu.sync_copy(g_v, g_sh.at[pl.ds(s * zc, zc)])
    pltpu.sync_copy(g_v, s_sh.at[pl.ds(s * zc, zc)])
    pltpu.sync_copy(dinv_v.at[pl.ds(c * (zc // 2), zc // 2)],
                    dinv_out.at[pl.ds(s * zc + c * (zc // 2), zc // 2)])
    plsc.subcore_barrier()

    pltpu.async_copy(g_sh.at[isrc_v.at[0]], vals0_v, sem0)

    def step(jj, carry):
        j0 = 2 * jj
        pltpu.async_copy(g_sh.at[isrc_v.at[j0 + 1]], vals1_v, sem1)
        pltpu.make_async_copy(g_sh.at[isrc_v.at[j0]], vals0_v, sem0).wait()
        pltpu.sync_copy(vals0_v, s_sh.at[idst_v.at[j0]], add=True)

        @pl.when(j0 + 2 < ch)
        def _():
            pltpu.async_copy(g_sh.at[isrc_v.at[j0 + 2]], vals0_v, sem0)

        pltpu.make_async_copy(g_sh.at[isrc_v.at[j0 + 1]], vals1_v, sem1).wait()
        pltpu.sync_copy(vals1_v, s_sh.at[idst_v.at[j0 + 1]], add=True)
        return carry

    lax.fori_loop(0, ch // 2, step, 0)
    plsc.subcore_barrier()
    pltpu.sync_copy(s_sh.at[pl.ds(s * zc, zc)],
                    s_out.at[c, pl.ds(s * zc, zc)])


def _h_body(x_ref, w1_ref, hp_ref):
    hmat = jnp.dot(x_ref[...], w1_ref[...], preferred_element_type=jnp.float32)
    colmask = (lax.broadcasted_iota(jnp.int32, hmat.shape, 1) == 3)
    hp_ref[...] = hmat + colmask.astype(jnp.float32)


def _out_body(s0_ref, s1_ref, hp_ref, dinv_ref, w2_ref, b1_ref, b2_ref, o_ref):
    dinv = dinv_ref[...][:, None]
    g = dinv * hp_ref[...]
    st = s0_ref[...] + s1_ref[...] - g
    acc = dinv * st + b1_ref[...]
    o_ref[...] = jnp.dot(acc, w2_ref[...],
                         preferred_element_type=jnp.float32) + b2_ref[...]


def kernel(x, edge_index, W1, b1, W2, b2):
    n, f = x.shape
    h = W1.shape[1]
    cdim = W2.shape[1]
    e = edge_index.shape[1]

    blk = 1024
    n_pad = -(-n // blk) * blk
    grid = n_pad // blk

    per_tile = -(-e // _NW)
    pt = -(-per_tile // (2 * _IDX)) * (2 * _IDX)
    ch = pt // _IDX
    e_pad = _NW * pt
    per_sub = -(-e // _NS)
    ptd = -(-per_sub // _IDX) * _IDX
    chd = ptd // _IDX
    e_padd = _NS * ptd

    src = edge_index[0].astype(jnp.int32)
    dst = edge_index[1].astype(jnp.int32)
    src_p = jnp.concatenate(
        [src, jnp.zeros((e_pad - e,), jnp.int32)]).reshape(_NW, ch, _IDX)
    dst_p = jnp.concatenate(
        [dst, jnp.full((e_pad - e,), n, jnp.int32)]).reshape(_NW, ch, _IDX)
    dst_d = jnp.concatenate(
        [dst, jnp.full((e_padd - e,), n, jnp.int32)]).reshape(_NS, chd, _IDX)

    x_p = jnp.pad(x, ((0, n_pad - n), (0, 0)))
    w1_p = jnp.pad(W1, ((0, 0), (0, _D - h)))
    w2_p = jnp.pad(W2, ((0, _D - h), (0, 0)))
    b1_p = jnp.pad(b1, (0, _D - h)).reshape(1, _D)
    b2_p = b2.reshape(1, cdim)

    hp_arr = pl.pallas_call(
        _h_body,
        grid=(grid,),
        in_specs=[
            pl.BlockSpec((blk, f), lambda i: (i, 0)),
            pl.BlockSpec((f, _D), lambda i: (0, 0)),
        ],
        out_specs=pl.BlockSpec((blk, _D), lambda i: (i, 0)),
        out_shape=jax.ShapeDtypeStruct((n_pad, _D), jnp.float32),
    )(x_p, w1_p)

    zc = n_pad // _NS
    fused_call = functools.partial(
        pl.kernel,
        out_type=(
            jax.ShapeDtypeStruct((_NC, n_pad, _D), jnp.float32),
            jax.ShapeDtypeStruct((n_pad,), jnp.float32),
        ),
        mesh=_mesh(),
        scratch_types=[
            pltpu.VMEM((chd, _IDX), jnp.int32),
            pltpu.VMEM((ch, _IDX), jnp.int32),
            pltpu.VMEM((ch, _IDX), jnp.int32),
            pltpu.VMEM((_IDX,), jnp.float32),
            pltpu.VMEM((zc,), jnp.float32),
            pltpu.VMEM((zc, _D), jnp.float32),
            pltpu.VMEM((zc, _D), jnp.float32),
            pltpu.VMEM((zc,), jnp.float32),
            pltpu.VMEM((zc,), jnp.float32),
            pltpu.VMEM((_IDX, _D), jnp.float32),
            pltpu.VMEM((_IDX, _D), jnp.float32),
            pltpu.VMEM_SHARED((n_pad,), jnp.float32),
            pltpu.VMEM_SHARED((n_pad, _D), jnp.float32),
            pltpu.VMEM_SHARED((n_pad, _D), jnp.float32),
            pltpu.SemaphoreType.DMA,
            pltpu.SemaphoreType.DMA,
        ],
        compiler_params=_SC_PARAMS,
    )(functools.partial(_fused_body, n_pad, chd, ch))
    s_part, dinv_arr = fused_call(dst_d, src_p, dst_p, hp_arr)

    out = pl.pallas_call(
        _out_body,
        grid=(grid,),
        in_specs=[
            pl.BlockSpec((blk, _D), lambda i: (i, 0)),
            pl.BlockSpec((blk, _D), lambda i: (i, 0)),
            pl.BlockSpec((blk, _D), lambda i: (i, 0)),
            pl.BlockSpec((blk,), lambda i: (i,)),
            pl.BlockSpec((_D, cdim), lambda i: (0, 0)),
            pl.BlockSpec((1, _D), lambda i: (0, 0)),
            pl.BlockSpec((1, cdim), lambda i: (0, 0)),
        ],
        out_specs=pl.BlockSpec((blk, cdim), lambda i: (i, 0)),
        out_shape=jax.ShapeDtypeStruct((n_pad, cdim), jnp.float32),
    )(s_part[0], s_part[1], hp_arr, dinv_arr, w2_p, b1_p, b2_p)

    return out[:n]


# speedup vs baseline: 74.3573x; 1.2700x over previous
import functools

import jax
import jax.numpy as jnp
from jax import lax
from jax.experimental import pallas as pl
from jax.experimental.pallas import tpu as pltpu
from jax.experimental.pallas import tpu_sc as plsc

_NC = 2
_NS = 16
_NW = _NC * _NS
_IDX = 128
_D = 16


def _mesh():
    return plsc.VectorSubcoreMesh(core_axis_name="c", subcore_axis_name="s")


_SC_PARAMS = pltpu.CompilerParams(use_tc_tiling_on_sc=False,
                                  needs_layout_passes=False)


def _newton_rsqrt(v):
    i = plsc.bitcast(v, jnp.int32)
    i = 0x5F3759DF - lax.shift_right_logical(i, 1)
    y = plsc.bitcast(i, jnp.float32)
    half = 0.5 * v
    for _ in range(3):
        y = y * (1.5 - half * y * y)
    return y


def _fused_body(n, n_pad, e_ch, eidx_hbm, hp_hbm, s_out, g_out,
                idxd_v, isrc_v, idst_v, ones_v, zero_v, hp_v, g_v,
                deg_v, dinv_v, vals0_v, vals1_v,
                deg_sh, g_sh, s_sh, sem0, sem1):
    c = lax.axis_index("c")
    s = lax.axis_index("s")
    w = c * _NS + s
    zc = n_pad // _NS
    hc = zc // 2

    base = e_ch // _NW
    rem = e_ch % _NW
    start = base * w + jnp.minimum(w, rem)
    extra = w < rem
    based = e_ch // _NS
    remd = e_ch % _NS
    startd = based * s + jnp.minimum(s, remd)
    extrad = s < remd

    for j in range(8):
        ones_v[pl.ds(16 * j, 16)] = jnp.ones((16,), jnp.float32)
    for j in range(zc // 16):
        zero_v[pl.ds(16 * j, 16)] = jnp.zeros((16,), jnp.float32)
    pltpu.sync_copy(zero_v, deg_sh.at[pl.ds(s * zc, zc)])
    pltpu.sync_copy(eidx_hbm.at[1, pl.ds(startd, based)],
                    idxd_v.at[pl.ds(0, based)])
    pltpu.sync_copy(eidx_hbm.at[0, pl.ds(start, base)],
                    isrc_v.at[pl.ds(0, base)])
    pltpu.sync_copy(eidx_hbm.at[1, pl.ds(start, base)],
                    idst_v.at[pl.ds(0, base)])

    @pl.when(extrad)
    def _():
        pltpu.sync_copy(eidx_hbm.at[1, pl.ds(startd + based, 1)],
                        idxd_v.at[pl.ds(based, 1)])

    @pl.when(extra)
    def _():
        pltpu.sync_copy(eidx_hbm.at[0, pl.ds(start + base, 1)],
                        isrc_v.at[pl.ds(base, 1)])
        pltpu.sync_copy(eidx_hbm.at[1, pl.ds(start + base, 1)],
                        idst_v.at[pl.ds(base, 1)])

    full_rows = n // zc
    tail_rows = n - full_rows * zc

    @pl.when(s < full_rows)
    def _():
        pltpu.sync_copy(hp_hbm.at[pl.ds(s * zc, zc)], hp_v)

    if tail_rows:
        @pl.when(s == full_rows)
        def _():
            pltpu.sync_copy(hp_hbm.at[pl.ds(full_rows * zc, tail_rows)],
                            hp_v.at[pl.ds(0, tail_rows)])

    plsc.subcore_barrier()

    def dstep(j, carry):
        pltpu.sync_copy(ones_v, deg_sh.at[idxd_v.at[j]], add=True)
        return carry

    lax.fori_loop(0, based + extrad.astype(jnp.int32), dstep, 0)
    plsc.subcore_barrier()

    pltpu.sync_copy(deg_sh.at[pl.ds(s * zc, zc)], deg_v)

    def nstep(k, carry):
        d16 = deg_v[pl.ds(16 * k, 16)] + 1.0
        dinv_v[pl.ds(16 * k, 16)] = _newton_rsqrt(d16)
        return carry

    lax.fori_loop(0, zc // 16, nstep, 0)

    def gstep(k, carry):
        d16 = dinv_v[pl.ds(16 * k, 16)]
        for j in range(16):
            r = 16 * k + j
            g_v[r, :] = hp_v[r, :] * d16[j]
        return carry

    lax.fori_loop(0, zc // 16, gstep, 0)
    pltpu.sync_copy(g_v, g_sh.at[pl.ds(s * zc, zc)])
    pltpu.sync_copy(g_v, s_sh.at[pl.ds(s * zc, zc)])
    pltpu.sync_copy(g_v.at[pl.ds(c * hc, hc)],
                    g_out.at[pl.ds(s * zc + c * hc, hc)])
    plsc.subcore_barrier()

    cnt = base + extra.astype(jnp.int32)
    pltpu.async_copy(g_sh.at[isrc_v.at[0]], vals0_v, sem0)

    def step(jj, carry):
        j0 = 2 * jj
        pltpu.async_copy(g_sh.at[isrc_v.at[j0 + 1]], vals1_v, sem1)
        pltpu.make_async_copy(g_sh.at[isrc_v.at[j0]], vals0_v, sem0).wait()
        pltpu.sync_copy(vals0_v, s_sh.at[idst_v.at[j0]], add=True)

        @pl.when(j0 + 2 < cnt)
        def _():
            pltpu.async_copy(g_sh.at[isrc_v.at[j0 + 2]], vals0_v, sem0)

        pltpu.make_async_copy(g_sh.at[isrc_v.at[j0 + 1]], vals1_v, sem1).wait()
        pltpu.sync_copy(vals1_v, s_sh.at[idst_v.at[j0 + 1]], add=True)
        return carry

    lax.fori_loop(0, base // 2, step, 0)

    @pl.when(cnt % 2 == 1)
    def _():
        j0 = cnt - 1
        pltpu.make_async_copy(g_sh.at[isrc_v.at[j0]], vals0_v, sem0).wait()
        pltpu.sync_copy(vals0_v, s_sh.at[idst_v.at[j0]], add=True)

    plsc.subcore_barrier()
    pltpu.sync_copy(s_sh.at[pl.ds(s * zc, zc)],
                    s_out.at[c, pl.ds(s * zc, zc)])


def _h_body(x_ref, w1_ref, hp_ref):
    hmat = jnp.dot(x_ref[...], w1_ref[...], preferred_element_type=jnp.float32)
    colmask = (lax.broadcasted_iota(jnp.int32, hmat.shape, 1) == 3)
    hp_ref[...] = hmat + colmask.astype(jnp.float32)


def _out_body(s0_ref, s1_ref, g_ref, w2_ref, b1_ref, b2_ref, o_ref):
    g = g_ref[...]
    st = s0_ref[0] + s1_ref[0] - g
    acc = g[:, 3:4] * st + b1_ref[...]
    o_ref[...] = jnp.dot(acc, w2_ref[...],
                         preferred_element_type=jnp.float32) + b2_ref[...]


def kernel(x, edge_index, W1, b1, W2, b2):
    n, f = x.shape
    h = W1.shape[1]
    cdim = W2.shape[1]
    e = edge_index.shape[1]

    n_pad = -(-n // (2 * _NS)) * (2 * _NS)
    n_pad = -(-n_pad // 512) * 512
    zc = n_pad // _NS

    eidx = edge_index.astype(jnp.int32)
    e_ch = e // _IDX
    if e % _IDX:
        e_ch += 1
        pad = e_ch * _IDX - e
        eidx = jnp.concatenate(
            [eidx, jnp.stack([jnp.zeros((pad,), jnp.int32),
                              jnp.full((pad,), n, jnp.int32)])], axis=1)
    eidx = eidx.reshape(2, e_ch, _IDX)

    w1_p = jnp.pad(W1, ((0, 0), (0, _D - h)))
    w2_p = jnp.pad(W2, ((0, _D - h), (0, 0)))
    b1_p = jnp.pad(b1, (0, _D - h)).reshape(1, _D)
    b2_p = b2.reshape(1, cdim)

    hblk = 2000
    hp_arr = pl.pallas_call(
        _h_body,
        grid=(n // hblk,),
        in_specs=[
            pl.BlockSpec((hblk, f), lambda i: (i, 0)),
            pl.BlockSpec((f, _D), lambda i: (0, 0)),
        ],
        out_specs=pl.BlockSpec((hblk, _D), lambda i: (i, 0)),
        out_shape=jax.ShapeDtypeStruct((n, _D), jnp.float32),
    )(x, w1_p)

    based = e_ch // _NS
    fused_call = functools.partial(
        pl.kernel,
        out_type=(
            jax.ShapeDtypeStruct((_NC, n_pad, _D), jnp.float32),
            jax.ShapeDtypeStruct((n_pad, _D), jnp.float32),
        ),
        mesh=_mesh(),
        scratch_types=[
            pltpu.VMEM((based + 1, _IDX), jnp.int32),
            pltpu.VMEM((e_ch // _NW + 1, _IDX), jnp.int32),
            pltpu.VMEM((e_ch // _NW + 1, _IDX), jnp.int32),
            pltpu.VMEM((_IDX,), jnp.float32),
            pltpu.VMEM((zc,), jnp.float32),
            pltpu.VMEM((zc, _D), jnp.float32),
            pltpu.VMEM((zc, _D), jnp.float32),
            pltpu.VMEM((zc,), jnp.float32),
            pltpu.VMEM((zc,), jnp.float32),
            pltpu.VMEM((_IDX, _D), jnp.float32),
            pltpu.VMEM((_IDX, _D), jnp.float32),
            pltpu.VMEM_SHARED((n_pad,), jnp.float32),
            pltpu.VMEM_SHARED((n_pad, _D), jnp.float32),
            pltpu.VMEM_SHARED((n_pad, _D), jnp.float32),
            pltpu.SemaphoreType.DMA,
            pltpu.SemaphoreType.DMA,
        ],
        compiler_params=_SC_PARAMS,
    )(functools.partial(_fused_body, n, n_pad, e_ch))
    s_part, g_arr = fused_call(eidx, hp_arr)

    out = pl.pallas_call(
        _out_body,
        grid=(1,),
        in_specs=[
            pl.BlockSpec((1, n, _D), lambda i: (0, 0, 0)),
            pl.BlockSpec((1, n, _D), lambda i: (1, 0, 0)),
            pl.BlockSpec((n, _D), lambda i: (0, 0)),
            pl.BlockSpec((_D, cdim), lambda i: (0, 0)),
            pl.BlockSpec((1, _D), lambda i: (0, 0)),
            pl.BlockSpec((1, cdim), lambda i: (0, 0)),
        ],
        out_specs=pl.BlockSpec((n, cdim), lambda i: (0, 0)),
        out_shape=jax.ShapeDtypeStruct((n, cdim), jnp.float32),
    )(s_part, s_part, g_arr, w2_p, b1_p, b2_p)

    return out


# speedup vs baseline: 77.8543x; 1.0470x over previous
import functools

import jax
import jax.numpy as jnp
from jax import lax
from jax.experimental import pallas as pl
from jax.experimental.pallas import tpu as pltpu
from jax.experimental.pallas import tpu_sc as plsc

_NC = 2
_NS = 16
_NW = _NC * _NS
_IDX = 128
_D = 16


def _mesh():
    return plsc.VectorSubcoreMesh(core_axis_name="c", subcore_axis_name="s")


_SC_PARAMS = pltpu.CompilerParams(use_tc_tiling_on_sc=False,
                                  needs_layout_passes=False)


def _newton_rsqrt(v):
    i = plsc.bitcast(v, jnp.int32)
    i = 0x5F3759DF - lax.shift_right_logical(i, 1)
    y = plsc.bitcast(i, jnp.float32)
    half = 0.5 * v
    for _ in range(3):
        y = y * (1.5 - half * y * y)
    return y


def _fused_body(n, n_pad, e_ch, eidx_hbm, hp_hbm, s_out, g_out,
                idxd_v, isrc_v, idst_v, zero_v, idxl_v, hist_v, hp_v, g_v,
                deg_v, dinv_v, vals0_v, vals1_v,
                deg_sh, g_sh, s_sh, sem0, sem1):
    c = lax.axis_index("c")
    s = lax.axis_index("s")
    w = c * _NS + s
    zc = n_pad // _NS
    hc = zc // 2
    zr = zc // 16
    nr = n_pad // 16

    base = e_ch // _NW
    rem = e_ch % _NW
    start = base * w + jnp.minimum(w, rem)
    extra = w < rem
    based = e_ch // _NS
    remd = e_ch % _NS
    startd = based * s + jnp.minimum(s, remd)
    extrad = s < remd

    iota16 = lax.broadcasted_iota(jnp.int32, (16,), 0)
    for j in range(zr):
        zero_v[j, :] = jnp.zeros((16,), jnp.float32)
    for r in range(nr // _IDX):
        for j in range(_IDX // 16):
            idxl_v[r, pl.ds(16 * j, 16)] = iota16 + (_IDX * r + 16 * j)
    pltpu.sync_copy(zero_v, deg_sh.at[pl.ds(s * zr, zr)])

    def zstep(r, carry):
        hist_v[r, :] = jnp.zeros((16,), jnp.float32)
        return carry

    lax.fori_loop(0, nr, zstep, 0)
    pltpu.sync_copy(eidx_hbm.at[1, pl.ds(startd, based)],
                    idxd_v.at[pl.ds(0, based)])
    pltpu.sync_copy(eidx_hbm.at[0, pl.ds(start, base)],
                    isrc_v.at[pl.ds(0, base)])
    pltpu.sync_copy(eidx_hbm.at[1, pl.ds(start, base)],
                    idst_v.at[pl.ds(0, base)])

    @pl.when(extrad)
    def _():
        pltpu.sync_copy(eidx_hbm.at[1, pl.ds(startd + based, 1)],
                        idxd_v.at[pl.ds(based, 1)])

    @pl.when(extra)
    def _():
        pltpu.sync_copy(eidx_hbm.at[0, pl.ds(start + base, 1)],
                        isrc_v.at[pl.ds(base, 1)])
        pltpu.sync_copy(eidx_hbm.at[1, pl.ds(start + base, 1)],
                        idst_v.at[pl.ds(base, 1)])

    full_rows = n // zc
    tail_rows = n - full_rows * zc

    @pl.when(s < full_rows)
    def _():
        pltpu.sync_copy(hp_hbm.at[pl.ds(s * zc, zc)], hp_v)

    if tail_rows:
        @pl.when(s == full_rows)
        def _():
            pltpu.sync_copy(hp_hbm.at[pl.ds(full_rows * zc, tail_rows)],
                            hp_v.at[pl.ds(0, tail_rows)])

    plsc.subcore_barrier()

    ones16 = jnp.ones((16,), jnp.float32)

    def dstep(j, carry):
        for m in range(_IDX // 16):
            ii = idxd_v[j, pl.ds(16 * m, 16)]
            plsc.addupdate_scatter(
                hist_v, [lax.shift_right_logical(ii, 4),
                         lax.bitwise_and(ii, 15)], ones16)
        return carry

    lax.fori_loop(0, based + extrad.astype(jnp.int32), dstep, 0)
    for r in range(nr // _IDX):
        pltpu.sync_copy(hist_v.at[pl.ds(_IDX * r, _IDX)],
                        deg_sh.at[idxl_v.at[r]], add=True)
    plsc.subcore_barrier()

    pltpu.sync_copy(deg_sh.at[pl.ds(s * zr, zr)], deg_v)

    def nstep(k, carry):
        d16 = deg_v[k, :] + 1.0
        dinv_v[k, :] = _newton_rsqrt(d16)
        return carry

    lax.fori_loop(0, zr, nstep, 0)

    def gstep(k, carry):
        d16 = dinv_v[k, :]
        for j in range(16):
            r = 16 * k + j
            g_v[r, :] = hp_v[r, :] * d16[j]
        return carry

    lax.fori_loop(0, zc // 16, gstep, 0)
    pltpu.sync_copy(g_v, g_sh.at[pl.ds(s * zc, zc)])
    pltpu.sync_copy(g_v, s_sh.at[pl.ds(s * zc, zc)])
    pltpu.sync_copy(g_v.at[pl.ds(c * hc, hc)],
                    g_out.at[pl.ds(s * zc + c * hc, hc)])
    plsc.subcore_barrier()

    cnt = base + extra.astype(jnp.int32)
    pltpu.async_copy(g_sh.at[isrc_v.at[0]], vals0_v, sem0)

    def step(jj, carry):
        j0 = 2 * jj
        pltpu.async_copy(g_sh.at[isrc_v.at[j0 + 1]], vals1_v, sem1)
        pltpu.make_async_copy(g_sh.at[isrc_v.at[j0]], vals0_v, sem0).wait()
        pltpu.sync_copy(vals0_v, s_sh.at[idst_v.at[j0]], add=True)

        @pl.when(j0 + 2 < cnt)
        def _():
            pltpu.async_copy(g_sh.at[isrc_v.at[j0 + 2]], vals0_v, sem0)

        pltpu.make_async_copy(g_sh.at[isrc_v.at[j0 + 1]], vals1_v, sem1).wait()
        pltpu.sync_copy(vals1_v, s_sh.at[idst_v.at[j0 + 1]], add=True)
        return carry

    lax.fori_loop(0, base // 2, step, 0)

    @pl.when(cnt % 2 == 1)
    def _():
        j0 = cnt - 1
        pltpu.make_async_copy(g_sh.at[isrc_v.at[j0]], vals0_v, sem0).wait()
        pltpu.sync_copy(vals0_v, s_sh.at[idst_v.at[j0]], add=True)

    plsc.subcore_barrier()
    pltpu.sync_copy(s_sh.at[pl.ds(s * zc, zc)],
                    s_out.at[c, pl.ds(s * zc, zc)])


def _h_body(x_ref, w1_ref, hp_ref):
    hmat = jnp.dot(x_ref[...], w1_ref[...], preferred_element_type=jnp.float32)
    colmask = (lax.broadcasted_iota(jnp.int32, hmat.shape, 1) == 3)
    hp_ref[...] = hmat + colmask.astype(jnp.float32)


def _out_body(s0_ref, s1_ref, g_ref, w2_ref, b1_ref, b2_ref, o_ref):
    g = g_ref[...]
    st = s0_ref[0] + s1_ref[0] - g
    acc = g[:, 3:4] * st + b1_ref[...]
    o_ref[...] = jnp.dot(acc, w2_ref[...],
                         preferred_element_type=jnp.float32) + b2_ref[...]


def kernel(x, edge_index, W1, b1, W2, b2):
    n, f = x.shape
    h = W1.shape[1]
    cdim = W2.shape[1]
    e = edge_index.shape[1]

    n_pad = -(-n // (2 * _NS)) * (2 * _NS)
    n_pad = -(-n_pad // 512) * 512
    zc = n_pad // _NS

    eidx = edge_index.astype(jnp.int32)
    e_ch = e // _IDX
    if e % _IDX:
        e_ch += 1
        pad = e_ch * _IDX - e
        eidx = jnp.concatenate(
            [eidx, jnp.stack([jnp.zeros((pad,), jnp.int32),
                              jnp.full((pad,), n, jnp.int32)])], axis=1)
    eidx = eidx.reshape(2, e_ch, _IDX)

    w1_p = jnp.pad(W1, ((0, 0), (0, _D - h)))
    w2_p = jnp.pad(W2, ((0, _D - h), (0, 0)))
    b1_p = jnp.pad(b1, (0, _D - h)).reshape(1, _D)
    b2_p = b2.reshape(1, cdim)

    hblk = 2000
    hp_arr = pl.pallas_call(
        _h_body,
        grid=(n // hblk,),
        in_specs=[
            pl.BlockSpec((hblk, f), lambda i: (i, 0)),
            pl.BlockSpec((f, _D), lambda i: (0, 0)),
        ],
        out_specs=pl.BlockSpec((hblk, _D), lambda i: (i, 0)),
        out_shape=jax.ShapeDtypeStruct((n, _D), jnp.float32),
    )(x, w1_p)

    based = e_ch // _NS
    fused_call = functools.partial(
        pl.kernel,
        out_type=(
            jax.ShapeDtypeStruct((_NC, n_pad, _D), jnp.float32),
            jax.ShapeDtypeStruct((n_pad, _D), jnp.float32),
        ),
        mesh=_mesh(),
        scratch_types=[
            pltpu.VMEM((based + 1, _IDX), jnp.int32),
            pltpu.VMEM((e_ch // _NW + 1, _IDX), jnp.int32),
            pltpu.VMEM((e_ch // _NW + 1, _IDX), jnp.int32),
            pltpu.VMEM((zc // 16, 16), jnp.float32),
            pltpu.VMEM((n_pad // 16 // _IDX, _IDX), jnp.int32),
            pltpu.VMEM((n_pad // 16, 16), jnp.float32),
            pltpu.VMEM((zc, _D), jnp.float32),
            pltpu.VMEM((zc, _D), jnp.float32),
            pltpu.VMEM((zc // 16, 16), jnp.float32),
            pltpu.VMEM((zc // 16, 16), jnp.float32),
            pltpu.VMEM((_IDX, _D), jnp.float32),
            pltpu.VMEM((_IDX, _D), jnp.float32),
            pltpu.VMEM_SHARED((n_pad // 16, 16), jnp.float32),
            pltpu.VMEM_SHARED((n_pad, _D), jnp.float32),
            pltpu.VMEM_SHARED((n_pad, _D), jnp.float32),
            pltpu.SemaphoreType.DMA,
            pltpu.SemaphoreType.DMA,
        ],
        compiler_params=_SC_PARAMS,
    )(functools.partial(_fused_body, n, n_pad, e_ch))
    s_part, g_arr = fused_call(eidx, hp_arr)

    out = pl.pallas_call(
        _out_body,
        grid=(1,),
        in_specs=[
            pl.BlockSpec((1, n, _D), lambda i: (0, 0, 0)),
            pl.BlockSpec((1, n, _D), lambda i: (1, 0, 0)),
            pl.BlockSpec((n, _D), lambda i: (0, 0)),
            pl.BlockSpec((_D, cdim), lambda i: (0, 0)),
            pl.BlockSpec((1, _D), lambda i: (0, 0)),
            pl.BlockSpec((1, cdim), lambda i: (0, 0)),
        ],
        out_specs=pl.BlockSpec((n, cdim), lambda i: (0, 0)),
        out_shape=jax.ShapeDtypeStruct((n, cdim), jnp.float32),
    )(s_part, s_part, g_arr, w2_p, b1_p, b2_p)

    return out
